# baseline (speedup 1.0000x reference)
import functools

import jax
import jax.numpy as jnp
from jax import lax
from jax.experimental import pallas as pl
from jax.experimental.pallas import tpu as pltpu
from jax.experimental.pallas import tpu_sc as plsc

N_TERMS = 100000
EMB_DIM = 64
BATCH = 4096
N_POS = 50
N_NEG = 200
TEMPERATURE = 0.1
K_PAD = 256
GS = 16
NGROUP = BATCH // GS
CHUNK_K = 16
NCHUNK = K_PAD // CHUNK_K
ROWS_PER_CHUNK = CHUNK_K * GS


def _sc_dots(table, idx_all):
    info = plsc.get_sparse_core_info()
    nw = info.num_cores * info.num_subcores
    gpw = NGROUP // nw
    spw = gpw * GS

    mesh = plsc.VectorSubcoreMesh(core_axis_name="c", subcore_axis_name="s")

    @functools.partial(
        pl.kernel,
        mesh=mesh,
        compiler_params=pltpu.CompilerParams(use_tc_tiling_on_sc=False,
                                             needs_layout_passes=False),
        out_type=(
            jax.ShapeDtypeStruct((K_PAD, BATCH), jnp.float32),
            jax.ShapeDtypeStruct((K_PAD, BATCH), jnp.float32),
        ),
        scratch_types=[
            pltpu.VMEM((spw * K_PAD,), jnp.int32),
            pltpu.VMEM((spw * K_PAD,), jnp.int32),
            pltpu.VMEM((ROWS_PER_CHUNK, EMB_DIM), jnp.float32),
            pltpu.VMEM((ROWS_PER_CHUNK, EMB_DIM), jnp.float32),
            pltpu.VMEM((EMB_DIM * GS,), jnp.float32),
            pltpu.VMEM((K_PAD, GS), jnp.float32),
            pltpu.VMEM((K_PAD, GS), jnp.float32),
            pltpu.SemaphoreType.DMA,
            pltpu.SemaphoreType.DMA,
        ],
    )
    def k(table_hbm, idx_hbm, dots_hbm, nsq_hbm,
          idx_slab, stage_v, buf_a, buf_b, anch_v, dots_v, nsq_v,
          sem_a, sem_b):
        wid = lax.axis_index("s") * info.num_cores + lax.axis_index("c")
        iota16 = lax.broadcasted_iota(jnp.int32, (16,), 0)
        tchunks = gpw * NCHUNK

        pltpu.sync_copy(idx_hbm.at[pl.ds(wid * spw * K_PAD, spw * K_PAD)],
                        idx_slab)

        def stage_group(g, carry):
            gread = (g * GS + iota16) * K_PAD
            gwrite = g * (K_PAD * GS)
            for k0 in range(K_PAD):
                kvec = jnp.bitwise_and(iota16 + k0, K_PAD - 1)
                vals = plsc.load_gather(idx_slab, [gread + kvec])
                plsc.store_scatter(
                    stage_v, [gwrite + jnp.left_shift(kvec, 4) + iota16], vals)
            return carry

        lax.fori_loop(0, gpw, stage_group, 0)

        NSTREAM = 4
        SLEN = ROWS_PER_CHUNK // NSTREAM

        def start(t, rows_v, sem):
            for j in range(NSTREAM):
                pltpu.async_copy(
                    table_hbm.at[stage_v.at[pl.ds(t * ROWS_PER_CHUNK
                                                  + j * SLEN, SLEN)]],
                    rows_v.at[pl.ds(j * SLEN, SLEN)], sem)

        def wait_all(t, rows_v, sem):
            for j in range(NSTREAM):
                pltpu.make_async_copy(
                    table_hbm.at[stage_v.at[pl.ds(t * ROWS_PER_CHUNK
                                                  + j * SLEN, SLEN)]],
                    rows_v.at[pl.ds(j * SLEN, SLEN)], sem).wait()

        def process(t, rows_v):
            g_local = t // NCHUNK
            c = t % NCHUNK
            g = wid * gpw + g_local

            @pl.when(c == 0)
            def _():
                for d in range(EMB_DIM):
                    dcol = jnp.bitwise_and(iota16 + d, EMB_DIM - 1)
                    av = plsc.load_gather(rows_v, [iota16, dcol])
                    anch_v[pl.ds(d * GS, GS)] = av

            def kb_body(kb, carry3):
                row_bases = [iota16 + (kb * 8 + kk) * GS for kk in range(8)]
                dots = [jnp.zeros((16,), jnp.float32) for _ in range(8)]
                sqs = [jnp.zeros((16,), jnp.float32) for _ in range(8)]
                for d in range(EMB_DIM):
                    a_d = anch_v[pl.ds(d * GS, GS)]
                    dcol = jnp.bitwise_and(iota16 + d, EMB_DIM - 1)
                    for kk in range(8):
                        rv = plsc.load_gather(rows_v, [row_bases[kk], dcol])
                        dots[kk] = dots[kk] + rv * a_d
                        sqs[kk] = sqs[kk] + rv * rv
                for kk in range(8):
                    kpos = c * CHUNK_K + kb * 8 + kk
                    dots_v[kpos] = dots[kk]
                    nsq_v[kpos] = sqs[kk]
                return carry3

            lax.fori_loop(0, CHUNK_K // 8, kb_body, 0)

            @pl.when(c == NCHUNK - 1)
            def _():
                pltpu.sync_copy(dots_v, dots_hbm.at[:, pl.ds(g * GS, GS)])
                pltpu.sync_copy(nsq_v, nsq_hbm.at[:, pl.ds(g * GS, GS)])

        start(0, buf_a, sem_a)

        def body(i, carry):
            t0 = 2 * i
            start(t0 + 1, buf_b, sem_b)
            wait_all(t0, buf_a, sem_a)
            process(t0, buf_a)

            @pl.when(i < tchunks // 2 - 1)
            def _():
                start(t0 + 2, buf_a, sem_a)

            wait_all(t0 + 1, buf_b, sem_b)
            process(t0 + 1, buf_b)
            return carry

        lax.fori_loop(0, tchunks // 2, body, 0)

    return k(table, idx_all)


_CB = 1024


def _tc_loss_body(d_ref, q_ref, out_ref):
    i = pl.program_id(0)

    @pl.when(i == 0)
    def _():
        out_ref[...] = jnp.zeros((1, 1), jnp.float32)

    d = d_ref[:]
    q = q_ref[:]
    na2 = q[0:1, :]
    denom = jnp.maximum(jnp.sqrt(na2 * q), 1e-8)
    logits = (d / denom) / TEMPERATURE
    row = lax.broadcasted_iota(jnp.int32, (K_PAD, _CB), 0)
    is_pos = jnp.logical_and(row >= 1, row <= N_POS)
    is_valid = jnp.logical_and(row >= 1, row <= N_POS + N_NEG)
    bce = (jnp.maximum(logits, 0.0)
           - jnp.where(is_pos, logits, 0.0)
           + jnp.log1p(jnp.exp(-jnp.abs(logits))))
    contrib = jnp.sum(jnp.where(is_valid, bce, 0.0))
    out_ref[...] += jnp.full((1, 1), contrib / (BATCH * (N_POS + N_NEG)),
                             jnp.float32)


def _tc_loss(dots_t, nsq_t):
    out = pl.pallas_call(
        _tc_loss_body,
        grid=(BATCH // _CB,),
        in_specs=[pl.BlockSpec((K_PAD, _CB), lambda i: (0, i)),
                  pl.BlockSpec((K_PAD, _CB), lambda i: (0, i))],
        out_specs=pl.BlockSpec((1, 1), lambda i: (0, 0)),
        out_shape=jax.ShapeDtypeStruct((1, 1), jnp.float32),
    )(dots_t, nsq_t)
    return out[0, 0]


def kernel(anchor_idx, positive_indices, negative_indices, table):
    idx_all = jnp.concatenate(
        [anchor_idx[:, None].astype(jnp.int32),
         positive_indices.astype(jnp.int32),
         negative_indices.astype(jnp.int32),
         jnp.zeros((BATCH, K_PAD - 1 - N_POS - N_NEG), jnp.int32)],
        axis=1)
    dots_t, nsq_t = _sc_dots(table, idx_all.reshape(BATCH * K_PAD))
    return _tc_loss(dots_t, nsq_t)


# speedup vs baseline: 1.0151x; 1.0151x over previous
import functools

import jax
import jax.numpy as jnp
from jax import lax
from jax.experimental import pallas as pl
from jax.experimental.pallas import tpu as pltpu
from jax.experimental.pallas import tpu_sc as plsc

N_TERMS = 100000
EMB_DIM = 64
BATCH = 4096
N_POS = 50
N_NEG = 200
TEMPERATURE = 0.1
K_PAD = 256
GS = 16
NGROUP = BATCH // GS
CHUNK_K = 16
NCHUNK = K_PAD // CHUNK_K
ROWS_PER_CHUNK = CHUNK_K * GS
HDIM = EMB_DIM // 2


def _sc_dots(table_i32, idx_all):
    info = plsc.get_sparse_core_info()
    nw = info.num_cores * info.num_subcores
    gpw = NGROUP // nw
    spw = gpw * GS

    mesh = plsc.VectorSubcoreMesh(core_axis_name="c", subcore_axis_name="s")

    @functools.partial(
        pl.kernel,
        mesh=mesh,
        compiler_params=pltpu.CompilerParams(use_tc_tiling_on_sc=False,
                                             needs_layout_passes=False),
        out_type=(
            jax.ShapeDtypeStruct((K_PAD, BATCH), jnp.float32),
            jax.ShapeDtypeStruct((K_PAD, BATCH), jnp.float32),
        ),
        scratch_types=[
            pltpu.VMEM((spw * K_PAD,), jnp.int32),
            pltpu.VMEM((spw * K_PAD,), jnp.int32),
            pltpu.VMEM((ROWS_PER_CHUNK, HDIM), jnp.int32),
            pltpu.VMEM((ROWS_PER_CHUNK, HDIM), jnp.int32),
            pltpu.VMEM((EMB_DIM * GS,), jnp.float32),
            pltpu.VMEM((K_PAD, GS), jnp.float32),
            pltpu.VMEM((K_PAD, GS), jnp.float32),
            pltpu.SemaphoreType.DMA,
            pltpu.SemaphoreType.DMA,
        ],
    )
    def k(table_hbm, idx_hbm, dots_hbm, nsq_hbm,
          idx_slab, stage_v, buf_a, buf_b, anch_v, dots_v, nsq_v,
          sem_a, sem_b):
        wid = lax.axis_index("s") * info.num_cores + lax.axis_index("c")
        iota16 = lax.broadcasted_iota(jnp.int32, (16,), 0)
        tchunks = gpw * NCHUNK

        pltpu.sync_copy(idx_hbm.at[pl.ds(wid * spw * K_PAD, spw * K_PAD)],
                        idx_slab)

        def stage_group(g, carry):
            gread = (g * GS + iota16) * K_PAD
            gwrite = g * (K_PAD * GS)
            for k0 in range(K_PAD):
                kvec = jnp.bitwise_and(iota16 + k0, K_PAD - 1)
                vals = plsc.load_gather(idx_slab, [gread + kvec])
                plsc.store_scatter(
                    stage_v, [gwrite + jnp.left_shift(kvec, 4) + iota16], vals)
            return carry

        lax.fori_loop(0, gpw, stage_group, 0)

        NSTREAM = 4
        SLEN = ROWS_PER_CHUNK // NSTREAM

        def start(t, rows_v, sem):
            for j in range(NSTREAM):
                pltpu.async_copy(
                    table_hbm.at[stage_v.at[pl.ds(t * ROWS_PER_CHUNK
                                                  + j * SLEN, SLEN)]],
                    rows_v.at[pl.ds(j * SLEN, SLEN)], sem)

        def wait_all(t, rows_v, sem):
            for j in range(NSTREAM):
                pltpu.make_async_copy(
                    table_hbm.at[stage_v.at[pl.ds(t * ROWS_PER_CHUNK
                                                  + j * SLEN, SLEN)]],
                    rows_v.at[pl.ds(j * SLEN, SLEN)], sem).wait()

        def process(t, rows_v):
            g_local = t // NCHUNK
            c = t % NCHUNK
            g = wid * gpw + g_local

            @pl.when(c == 0)
            def _():
                for p in range(HDIM):
                    pcol = jnp.bitwise_and(iota16 + p, HDIM - 1)
                    av = plsc.load_gather(rows_v, [iota16, pcol])
                    alo = plsc.bitcast(jnp.left_shift(av, 16), jnp.float32)
                    ahi = plsc.bitcast(jnp.bitwise_and(av, jnp.int32(-65536)),
                                       jnp.float32)
                    anch_v[pl.ds(p * 2 * GS, GS)] = alo
                    anch_v[pl.ds((p * 2 + 1) * GS, GS)] = ahi

            def kb_body(kb, carry3):
                row_bases = [iota16 + (kb * 8 + kk) * GS for kk in range(8)]
                dots = [jnp.zeros((16,), jnp.float32) for _ in range(8)]
                sqs = [jnp.zeros((16,), jnp.float32) for _ in range(8)]
                for p in range(HDIM):
                    a_lo = anch_v[pl.ds(p * 2 * GS, GS)]
                    a_hi = anch_v[pl.ds((p * 2 + 1) * GS, GS)]
                    pcol = jnp.bitwise_and(iota16 + p, HDIM - 1)
                    for kk in range(8):
                        rv = plsc.load_gather(rows_v, [row_bases[kk], pcol])
                        lo = plsc.bitcast(jnp.left_shift(rv, 16), jnp.float32)
                        hi = plsc.bitcast(jnp.bitwise_and(rv, jnp.int32(-65536)),
                                          jnp.float32)
                        dots[kk] = dots[kk] + lo * a_lo + hi * a_hi
                        sqs[kk] = sqs[kk] + lo * lo + hi * hi
                for kk in range(8):
                    kpos = c * CHUNK_K + kb * 8 + kk
                    dots_v[kpos] = dots[kk]
                    nsq_v[kpos] = sqs[kk]
                return carry3

            lax.fori_loop(0, CHUNK_K // 8, kb_body, 0)

            @pl.when(c == NCHUNK - 1)
            def _():
                pltpu.sync_copy(dots_v, dots_hbm.at[:, pl.ds(g * GS, GS)])
                pltpu.sync_copy(nsq_v, nsq_hbm.at[:, pl.ds(g * GS, GS)])

        start(0, buf_a, sem_a)

        def body(i, carry):
            t0 = 2 * i
            start(t0 + 1, buf_b, sem_b)
            wait_all(t0, buf_a, sem_a)
            process(t0, buf_a)

            @pl.when(i < tchunks // 2 - 1)
            def _():
                start(t0 + 2, buf_a, sem_a)

            wait_all(t0 + 1, buf_b, sem_b)
            process(t0 + 1, buf_b)
            return carry

        lax.fori_loop(0, tchunks // 2, body, 0)

    return k(table_i32, idx_all)


_CB = 1024


def _tc_loss_body(d_ref, q_ref, out_ref):
    i = pl.program_id(0)

    @pl.when(i == 0)
    def _():
        out_ref[...] = jnp.zeros((1, 1), jnp.float32)

    d = d_ref[:]
    q = q_ref[:]
    na2 = q[0:1, :]
    denom = jnp.maximum(jnp.sqrt(na2 * q), 1e-8)
    logits = (d / denom) / TEMPERATURE
    row = lax.broadcasted_iota(jnp.int32, (K_PAD, _CB), 0)
    is_pos = jnp.logical_and(row >= 1, row <= N_POS)
    is_valid = jnp.logical_and(row >= 1, row <= N_POS + N_NEG)
    bce = (jnp.maximum(logits, 0.0)
           - jnp.where(is_pos, logits, 0.0)
           + jnp.log1p(jnp.exp(-jnp.abs(logits))))
    contrib = jnp.sum(jnp.where(is_valid, bce, 0.0))
    out_ref[...] += jnp.full((1, 1), contrib / (BATCH * (N_POS + N_NEG)),
                             jnp.float32)


def _tc_loss(dots_t, nsq_t):
    out = pl.pallas_call(
        _tc_loss_body,
        grid=(BATCH // _CB,),
        in_specs=[pl.BlockSpec((K_PAD, _CB), lambda i: (0, i)),
                  pl.BlockSpec((K_PAD, _CB), lambda i: (0, i))],
        out_specs=pl.BlockSpec((1, 1), lambda i: (0, 0)),
        out_shape=jax.ShapeDtypeStruct((1, 1), jnp.float32),
    )(dots_t, nsq_t)
    return out[0, 0]


def kernel(anchor_idx, positive_indices, negative_indices, table):
    idx_all = jnp.concatenate(
        [anchor_idx[:, None].astype(jnp.int32),
         positive_indices.astype(jnp.int32),
         negative_indices.astype(jnp.int32),
         jnp.zeros((BATCH, K_PAD - 1 - N_POS - N_NEG), jnp.int32)],
        axis=1)
    table_i32 = lax.bitcast_convert_type(
        table.astype(jnp.bfloat16).reshape(N_TERMS, HDIM, 2), jnp.int32)
    dots_t, nsq_t = _sc_dots(table_i32, idx_all.reshape(BATCH * K_PAD))
    return _tc_loss(dots_t, nsq_t)


# speedup vs baseline: 1.0481x; 1.0324x over previous
import functools
import jax
import jax.numpy as jnp
from jax import lax
from jax.experimental import pallas as pl
from jax.experimental.pallas import tpu as pltpu
from jax.experimental.pallas import tpu_sc as plsc

N_TERMS = 100000
EMB_DIM = 64
BATCH = 4096
N_POS = 50
N_NEG = 200
TEMPERATURE = 0.1
K_PAD = 256
GS = 16
NGROUP = BATCH // GS
CHUNK_K = 32
NCHUNK = K_PAD // CHUNK_K
ROWS_PER_CHUNK = CHUNK_K * GS
HDIM = EMB_DIM // 2


def _sc_dots(table_i32, idx_flat):
    info = plsc.get_sparse_core_info()
    nw = info.num_cores * info.num_subcores
    gpw = NGROUP // nw

    mesh = plsc.VectorSubcoreMesh(core_axis_name="c", subcore_axis_name="s")

    @functools.partial(
        pl.kernel,
        mesh=mesh,
        compiler_params=pltpu.CompilerParams(use_tc_tiling_on_sc=False,
                                             needs_layout_passes=False),
        out_type=(
            jax.ShapeDtypeStruct((K_PAD, BATCH), jnp.float32),
            jax.ShapeDtypeStruct((K_PAD, BATCH), jnp.float32),
        ),
        scratch_types=[
            pltpu.VMEM((gpw * K_PAD * GS,), jnp.int32),
            pltpu.VMEM((ROWS_PER_CHUNK, HDIM), jnp.int32),
            pltpu.VMEM((ROWS_PER_CHUNK, HDIM), jnp.int32),
            pltpu.VMEM((ROWS_PER_CHUNK, HDIM), jnp.int32),
            pltpu.VMEM((EMB_DIM * GS,), jnp.float32),
            pltpu.VMEM((K_PAD, GS), jnp.float32),
            pltpu.VMEM((K_PAD, GS), jnp.float32),
            pltpu.SemaphoreType.DMA,
            pltpu.SemaphoreType.DMA,
            pltpu.SemaphoreType.DMA,
        ],
    )
    def k(table_hbm, idx_hbm, dots_hbm, nsq_hbm,
          idx_slab, buf_a, buf_b, buf_c, anch_v, dots_v, nsq_v,
          sem_a, sem_b, sem_c):
        wid = lax.axis_index("s") * info.num_cores + lax.axis_index("c")
        iota16 = lax.broadcasted_iota(jnp.int32, (16,), 0)
        woff = wid * (gpw * K_PAD * GS)
        tchunks = gpw * NCHUNK

        pltpu.sync_copy(idx_hbm.at[pl.ds(woff, gpw * K_PAD * GS)], idx_slab)

        NSTREAM = 8
        SLEN = ROWS_PER_CHUNK // NSTREAM

        def start(t, rows_v, sem):
            for j in range(NSTREAM):
                pltpu.async_copy(
                    table_hbm.at[idx_slab.at[pl.ds(t * ROWS_PER_CHUNK
                                                   + j * SLEN, SLEN)]],
                    rows_v.at[pl.ds(j * SLEN, SLEN)], sem)

        def wait_all(t, rows_v, sem):
            for j in range(NSTREAM):
                pltpu.make_async_copy(
                    table_hbm.at[idx_slab.at[pl.ds(t * ROWS_PER_CHUNK
                                                   + j * SLEN, SLEN)]],
                    rows_v.at[pl.ds(j * SLEN, SLEN)], sem).wait()

        def process(t, rows_v):
            g_local = t // NCHUNK
            c = t % NCHUNK
            g = wid * gpw + g_local

            @pl.when(c == 0)
            def _():
                for p in range(HDIM):
                    pcol = jnp.bitwise_and(iota16 + p, HDIM - 1)
                    av = plsc.load_gather(rows_v, [iota16, pcol])
                    alo = plsc.bitcast(jnp.left_shift(av, 16), jnp.float32)
                    ahi = plsc.bitcast(jnp.bitwise_and(av, jnp.int32(-65536)),
                                       jnp.float32)
                    anch_v[pl.ds(p * 2 * GS, GS)] = alo
                    anch_v[pl.ds((p * 2 + 1) * GS, GS)] = ahi

            def kb_body(kb, carry3):
                row_bases = [iota16 + (kb * 8 + kk) * GS for kk in range(8)]
                dots = [jnp.zeros((16,), jnp.float32) for _ in range(8)]
                sqs = [jnp.zeros((16,), jnp.float32) for _ in range(8)]
                for p in range(HDIM):
                    a_lo = anch_v[pl.ds(p * 2 * GS, GS)]
                    a_hi = anch_v[pl.ds((p * 2 + 1) * GS, GS)]
                    pcol = jnp.bitwise_and(iota16 + p, HDIM - 1)
                    for kk in range(8):
                        rv = plsc.load_gather(rows_v, [row_bases[kk], pcol])
                        lo = plsc.bitcast(jnp.left_shift(rv, 16), jnp.float32)
                        hi = plsc.bitcast(jnp.bitwise_and(rv, jnp.int32(-65536)),
                                          jnp.float32)
                        dots[kk] = dots[kk] + lo * a_lo + hi * a_hi
                        sqs[kk] = sqs[kk] + lo * lo + hi * hi
                for kk in range(8):
                    kpos = c * CHUNK_K + kb * 8 + kk
                    dots_v[kpos] = dots[kk]
                    nsq_v[kpos] = sqs[kk]
                return carry3

            lax.fori_loop(0, CHUNK_K // 8, kb_body, 0)

            @pl.when(c == NCHUNK - 1)
            def _():
                pltpu.sync_copy(dots_v, dots_hbm.at[:, pl.ds(g * GS, GS)])
                pltpu.sync_copy(nsq_v, nsq_hbm.at[:, pl.ds(g * GS, GS)])

        bufs = (buf_a, buf_b, buf_c)
        sems = (sem_a, sem_b, sem_c)
        start(0, buf_a, sem_a)
        start(1, buf_b, sem_b)

        def body(i, carry):
            for u in range(3):
                t = 3 * i + u

                @pl.when(t + 2 < tchunks)
                def _():
                    start(t + 2, bufs[(u + 2) % 3], sems[(u + 2) % 3])

                @pl.when(t < tchunks)
                def _():
                    wait_all(t, bufs[u], sems[u])
                    process(t, bufs[u])
            return carry

        lax.fori_loop(0, (tchunks + 2) // 3, body, 0)

    return k(table_i32, idx_flat)


_CB = 1024


def _tc_loss_body(d_ref, q_ref, out_ref):
    i = pl.program_id(0)

    @pl.when(i == 0)
    def _():
        out_ref[...] = jnp.zeros((1, 1), jnp.float32)

    d = d_ref[:]
    q = q_ref[:]
    na2 = q[0:1, :]
    denom = jnp.maximum(jnp.sqrt(na2 * q), 1e-8)
    logits = (d / denom) / TEMPERATURE
    row = lax.broadcasted_iota(jnp.int32, (K_PAD, _CB), 0)
    is_pos = jnp.logical_and(row >= 1, row <= N_POS)
    is_valid = jnp.logical_and(row >= 1, row <= N_POS + N_NEG)
    bce = (jnp.maximum(logits, 0.0)
           - jnp.where(is_pos, logits, 0.0)
           + jnp.log1p(jnp.exp(-jnp.abs(logits))))
    contrib = jnp.sum(jnp.where(is_valid, bce, 0.0))
    out_ref[...] += jnp.full((1, 1), contrib / (BATCH * (N_POS + N_NEG)),
                             jnp.float32)


def _tc_loss(dots_t, nsq_t):
    out = pl.pallas_call(
        _tc_loss_body,
        grid=(BATCH // _CB,),
        in_specs=[pl.BlockSpec((K_PAD, _CB), lambda i: (0, i)),
                  pl.BlockSpec((K_PAD, _CB), lambda i: (0, i))],
        out_specs=pl.BlockSpec((1, 1), lambda i: (0, 0)),
        out_shape=jax.ShapeDtypeStruct((1, 1), jnp.float32),
    )(dots_t, nsq_t)
    return out[0, 0]


def kernel(anchor_idx, positive_indices, negative_indices, table):
    idx_all = jnp.concatenate(
        [anchor_idx[:, None].astype(jnp.int32),
         positive_indices.astype(jnp.int32),
         negative_indices.astype(jnp.int32),
         jnp.zeros((BATCH, K_PAD - 1 - N_POS - N_NEG), jnp.int32)],
        axis=1)
    idx_gks = idx_all.reshape(NGROUP, GS, K_PAD).transpose(0, 2, 1)
    idx_flat = idx_gks.reshape(NGROUP * K_PAD * GS)
    table_i32 = lax.bitcast_convert_type(
        table.astype(jnp.bfloat16).reshape(N_TERMS, HDIM, 2), jnp.int32)
    dots_t, nsq_t = _sc_dots(table_i32, idx_flat)
    return _tc_loss(dots_t, nsq_t)
